# baseline (speedup 1.0000x reference)
import jax
import jax.numpy as jnp
from jax.experimental import pallas as pl

_NB = 1000


def _tc_body(q_ref, ent_in_ref, nb_ref, Bm_ref, S_ref, ET_ref, K_ref,
             WentT_ref, bent_ref, WprojT_ref, gbias_ref, WrankT_ref,
             brank_ref, c_ref, out_ref):
    ent = jnp.dot(ent_in_ref[...], WentT_ref[...],
                  preferred_element_type=jnp.float32) + bent_ref[...]
    qB = jnp.dot(q_ref[...], Bm_ref[...], preferred_element_type=jnp.float32)
    s_tgt = jnp.sum(qB * ent, axis=1, keepdims=True) + c_ref[0, 0]
    nb = nb_ref[...]
    s_src = jnp.dot(nb, S_ref[...], preferred_element_type=jnp.float32)
    s = s_src + s_tgt
    s = jnp.where(s > 0, s, 0.2 * s)
    e = jnp.exp(s)
    denom = jnp.sum(e, axis=1, keepdims=True) + 1e-16
    attn = e / denom
    attn_exp = jnp.dot(attn, ET_ref[...], preferred_element_type=jnp.float32)
    agg = jnp.dot(nb * attn_exp, K_ref[...],
                  preferred_element_type=jnp.float32)
    out_n = jnp.dot(agg, WprojT_ref[...],
                    preferred_element_type=jnp.float32) + gbias_ref[...]
    out_n = jnp.where(out_n > 0, out_n, jnp.exp(jnp.minimum(out_n, 0.0)) - 1.0)
    rank = jnp.dot(out_n, WrankT_ref[...],
                   preferred_element_type=jnp.float32) + brank_ref[...]
    out_ref[...] = rank


@jax.jit
def kernel(query_emb, entity_emb, neighbors, W_ent, b_ent, A_bil, b_bil,
           W_proj, a_src, a_tgt, gat_bias, W_rank, b_rank):
    N, deg, D = neighbors.shape
    ent_in = entity_emb.shape[1]
    q = jnp.squeeze(query_emb, axis=1)
    nb_flat = neighbors.reshape(N, deg * D)

    u = W_proj.T @ a_tgt
    v = W_proj.T @ a_src
    Bm = jnp.einsum('k,kij->ij', u, A_bil)
    c = jnp.dot(b_bil, u).reshape(1, 1)

    eye_d = jnp.eye(deg, dtype=jnp.float32)
    eye_k = jnp.eye(D, dtype=jnp.float32)
    S = jnp.kron(eye_d, v.reshape(D, 1))
    ET = jnp.kron(eye_d, jnp.ones((1, D), jnp.float32))
    K = jnp.kron(jnp.ones((deg, 1), jnp.float32), eye_k)

    nblk = _NB
    grid = N // nblk

    const = lambda shape: pl.BlockSpec(shape, lambda i: (0,) * len(shape))
    out = pl.pallas_call(
        _tc_body,
        grid=(grid,),
        in_specs=[
            pl.BlockSpec((nblk, D), lambda i: (i, 0)),
            pl.BlockSpec((nblk, ent_in), lambda i: (i, 0)),
            pl.BlockSpec((nblk, deg * D), lambda i: (i, 0)),
            const((D, D)),
            const((deg * D, deg)),
            const((deg, deg * D)),
            const((deg * D, D)),
            const((ent_in, D)),
            const((1, D)),
            const((D, D)),
            const((1, D)),
            const((D, 1)),
            const((1, 1)),
            const((1, 1)),
        ],
        out_specs=pl.BlockSpec((nblk, 1), lambda i: (i, 0)),
        out_shape=jax.ShapeDtypeStruct((N, 1), jnp.float32),
    )(
        q, entity_emb, nb_flat, Bm, S, ET, K, W_ent.T,
        b_ent.reshape(1, D), W_proj.T, gat_bias.reshape(1, D), W_rank.T,
        b_rank.reshape(1, 1), c,
    )
    return out


# speedup vs baseline: 6.4704x; 6.4704x over previous
import jax
import jax.numpy as jnp
from jax.experimental import pallas as pl

_NB = 1024


def _tc_body(qT_ref, entT_ref, nbT_ref, BT_ref, v3_ref, Went_ref, bent_ref,
             Wproj_ref, gbias_ref, Wrank_ref, brank_ref, c_ref, out_ref):
    entp = jnp.dot(Went_ref[...], entT_ref[...],
                   preferred_element_type=jnp.float32) + bent_ref[...]
    qB = jnp.dot(BT_ref[...], qT_ref[...], preferred_element_type=jnp.float32)
    s_tgt = jnp.sum(qB * entp, axis=0, keepdims=True) + c_ref[0, 0]
    nb = nbT_ref[...]
    s_src = jnp.sum(nb * v3_ref[...], axis=0)
    s = s_src + s_tgt
    s = jnp.where(s > 0, s, 0.2 * s)
    e = jnp.exp(s)
    denom = jnp.sum(e, axis=0, keepdims=True) + 1e-16
    attn = e / denom
    agg = jnp.sum(nb * attn[None, :, :], axis=1)
    out_n = jnp.dot(Wproj_ref[...], agg,
                    preferred_element_type=jnp.float32) + gbias_ref[...]
    out_n = jnp.where(out_n > 0, out_n, jnp.exp(jnp.minimum(out_n, 0.0)) - 1.0)
    rank = jnp.dot(Wrank_ref[...], out_n,
                   preferred_element_type=jnp.float32) + brank_ref[...]
    out_ref[...] = rank


@jax.jit
def kernel(query_emb, entity_emb, neighbors, W_ent, b_ent, A_bil, b_bil,
           W_proj, a_src, a_tgt, gat_bias, W_rank, b_rank):
    N, deg, D = neighbors.shape
    ent_in = entity_emb.shape[1]
    qT = jnp.squeeze(query_emb, axis=1).T
    entT = entity_emb.T
    nbT = jnp.transpose(neighbors, (2, 1, 0))

    u = W_proj.T @ a_tgt
    v = W_proj.T @ a_src
    BT = jnp.einsum('k,kij->ji', u, A_bil)
    c = jnp.dot(b_bil, u).reshape(1, 1)

    nblk = _NB
    grid = pl.cdiv(N, nblk)

    const = lambda shape: pl.BlockSpec(shape, lambda i: (0,) * len(shape))
    outT = pl.pallas_call(
        _tc_body,
        grid=(grid,),
        in_specs=[
            pl.BlockSpec((D, nblk), lambda i: (0, i)),
            pl.BlockSpec((ent_in, nblk), lambda i: (0, i)),
            pl.BlockSpec((D, deg, nblk), lambda i: (0, 0, i)),
            const((D, D)),
            const((D, 1, 1)),
            const((D, ent_in)),
            const((D, 1)),
            const((D, D)),
            const((D, 1)),
            const((1, D)),
            const((1, 1)),
            const((1, 1)),
        ],
        out_specs=pl.BlockSpec((1, nblk), lambda i: (0, i)),
        out_shape=jax.ShapeDtypeStruct((1, N), jnp.float32),
    )(
        qT, entT, nbT, BT, v.reshape(D, 1, 1), W_ent,
        b_ent.reshape(D, 1), W_proj, gat_bias.reshape(D, 1), W_rank,
        b_rank.reshape(1, 1), c,
    )
    return outT.T
